# baseline (speedup 1.0000x reference)
import functools

import jax
import jax.numpy as jnp
from jax import lax
from jax.experimental import pallas as pl
from jax.experimental.pallas import tpu as pltpu
from jax.experimental.pallas import tpu_sc as plsc

_N = 10000
_E = 320000
_DIN = 128
_H = 64
_G = 128
_OUT = 2

_NC = 2
_NS = 16
_NW = _NC * _NS
_EPW = _E // _NW
_C = 100
_NCHUNK = _EPW // _C
_NB = 4
_RPT = 632
_RLAST = _N - (_NS - 1) * _RPT
_DW = 2
_DVW = 8

_ROWS = 1000
_NBLK = _N // _ROWS

_sc_mesh = plsc.VectorSubcoreMesh(core_axis_name="c", subcore_axis_name="s",
                                  num_cores=_NC, num_subcores=_NS)
_sc_params = pltpu.CompilerParams(use_tc_tiling_on_sc=False)


_DEG_WIN = 8


@functools.partial(
    pl.kernel,
    out_type=jax.ShapeDtypeStruct((_NC, _N, _DW), jnp.float32),
    mesh=_sc_mesh,
    scratch_types=[
        pltpu.VMEM_SHARED((_N, _DW), jnp.float32),
        pltpu.VMEM((_C, _DW), jnp.float32),
        pltpu.VMEM((_NCHUNK, _C), jnp.int32),
        pltpu.SemaphoreType.DMA,
    ],
    compiler_params=_sc_params,
)
def _sc_degree(dst3_hbm, ones_hbm, zeros_hbm, out_hbm, dacc, ones_v, dst_v,
               ssem):
    c = lax.axis_index("c")
    s = lax.axis_index("s")
    wid = s * _NC + c

    idx_cp = pltpu.async_copy(dst3_hbm.at[wid], dst_v, ssem)
    pltpu.sync_copy(ones_hbm, ones_v)

    @pl.when(s < _NS - 1)
    def _():
        pltpu.sync_copy(zeros_hbm.at[pl.ds(s * _RPT, _RPT)],
                        dacc.at[pl.ds(s * _RPT, _RPT)])

    @pl.when(s == _NS - 1)
    def _():
        pltpu.sync_copy(zeros_hbm.at[pl.ds(s * _RPT, _RLAST)],
                        dacc.at[pl.ds(s * _RPT, _RLAST)])

    idx_cp.wait()
    plsc.subcore_barrier()

    def body(j, carry):
        pltpu.async_copy(ones_v, dacc.at[dst_v.at[j]], ssem, add=True)

        @pl.when(j >= _DEG_WIN)
        def _():
            pltpu.make_async_copy(ones_v, dacc.at[dst_v.at[j - _DEG_WIN]],
                                  ssem).wait()

        return carry

    lax.fori_loop(0, _NCHUNK, body, 0)
    for k in range(_DEG_WIN):
        pltpu.make_async_copy(ones_v, dacc.at[dst_v.at[_NCHUNK - _DEG_WIN + k]],
                              ssem).wait()
    plsc.subcore_barrier()

    @pl.when(s < _NS - 1)
    def _():
        pltpu.sync_copy(dacc.at[pl.ds(s * _RPT, _RPT)],
                        out_hbm.at[c, pl.ds(s * _RPT, _RPT)])

    @pl.when(s == _NS - 1)
    def _():
        pltpu.sync_copy(dacc.at[pl.ds(s * _RPT, _RLAST)],
                        out_hbm.at[c, pl.ds(s * _RPT, _RLAST)])


@functools.partial(
    pl.kernel,
    out_type=jax.ShapeDtypeStruct((_NC, _N, _H), jnp.float32),
    mesh=_sc_mesh,
    scratch_types=[
        pltpu.VMEM_SHARED((_N, _H), jnp.float32),
        pltpu.VMEM((_NCHUNK, _C), jnp.int32),
        pltpu.VMEM((_NCHUNK, _C), jnp.int32),
        pltpu.VMEM((_NB, _C, _H), jnp.float32),
        pltpu.VMEM((_RPT, _H), jnp.float32),
    ] + [pltpu.SemaphoreType.DMA] * (2 * _NB),
    compiler_params=_sc_params,
)
def _sc_aggregate(g_hbm, src3_hbm, dst3_hbm, out_hbm, acc, src_v,
                  dst_v, rows_v, zbuf, *sems):
    gsem = sems[:_NB]
    ssem = sems[_NB:]
    c = lax.axis_index("c")
    s = lax.axis_index("s")
    wid = s * _NC + c

    src_cp = pltpu.async_copy(src3_hbm.at[wid], src_v, sems[0])
    dst_cp = pltpu.async_copy(dst3_hbm.at[wid], dst_v, sems[1])

    def zfill(i, carry):
        for k in range(_H // 16):
            zbuf[i, pl.ds(16 * k, 16)] = jnp.zeros((16,), jnp.float32)
        return carry

    lax.fori_loop(0, _RPT, zfill, 0)

    @pl.when(s < _NS - 1)
    def _():
        pltpu.sync_copy(zbuf, acc.at[pl.ds(s * _RPT, _RPT)])

    @pl.when(s == _NS - 1)
    def _():
        pltpu.sync_copy(zbuf.at[pl.ds(0, _RLAST)],
                        acc.at[pl.ds(s * _RPT, _RLAST)])

    src_cp.wait()
    dst_cp.wait()
    plsc.subcore_barrier()

    def g_issue(b, j):
        pltpu.async_copy(g_hbm.at[src_v.at[j]], rows_v.at[b], gsem[b])

    def g_wait(b, j):
        pltpu.make_async_copy(g_hbm.at[src_v.at[j]], rows_v.at[b],
                              gsem[b]).wait()

    def s_issue(b, j):
        pltpu.async_copy(rows_v.at[b], acc.at[dst_v.at[j]], ssem[b], add=True)

    def s_wait(b, j):
        pltpu.make_async_copy(rows_v.at[b], acc.at[dst_v.at[j]],
                              ssem[b]).wait()

    for b in range(_NB - 1):
        g_issue(b, b)
    g_wait(0, 0)
    s_issue(0, 0)
    g_issue(_NB - 1, _NB - 1)
    for b in range(1, _NB):
        g_wait(b, b)
        s_issue(b, b)
        b3 = (b + _NB - 1) % _NB
        s_wait(b3, b - 1)
        g_issue(b3, b + _NB - 1)

    def body(j0, carry):
        base = j0 * _NB
        for b in range(_NB):
            j = base + b
            g_wait(b, j)
            s_issue(b, j)
            b3 = (b + _NB - 1) % _NB
            s_wait(b3, j - 1)
            g_issue(b3, j + _NB - 1)
        return carry

    lax.fori_loop(1, _NCHUNK // _NB - 1, body, 0)

    base = _NCHUNK - _NB
    g_wait(0, base)
    s_issue(0, base)
    s_wait(_NB - 1, base - 1)
    g_issue(_NB - 1, _NCHUNK - 1)
    for b in range(1, _NB):
        g_wait(b, base + b)
        s_issue(b, base + b)
    for b in range(_NB):
        s_wait(b, base + b)

    plsc.subcore_barrier()

    @pl.when(s < _NS - 1)
    def _():
        pltpu.sync_copy(acc.at[pl.ds(s * _RPT, _RPT)],
                        out_hbm.at[c, pl.ds(s * _RPT, _RPT)])

    @pl.when(s == _NS - 1)
    def _():
        pltpu.sync_copy(acc.at[pl.ds(s * _RPT, _RLAST)],
                        out_hbm.at[c, pl.ds(s * _RPT, _RLAST)])


def _mm_scale_body(deg_ref, x_ref, w_ref, g_ref, dinv_ref):
    hw = jnp.dot(x_ref[...], w_ref[...], preferred_element_type=jnp.float32)
    deg = deg_ref[0, :, 0:1] + deg_ref[1, :, 0:1] + 1.0
    dinv = lax.rsqrt(jnp.maximum(deg, 1.0))
    dinv_ref[...] = jnp.broadcast_to(dinv, (_ROWS, _DVW))
    g_ref[...] = hw * dinv


def _mm_scale(deg_parts, x, w):
    k = x.shape[1]
    return pl.pallas_call(
        _mm_scale_body,
        grid=(_NBLK,),
        in_specs=[
            pl.BlockSpec((_NC, _ROWS, _DW), lambda i: (0, i, 0)),
            pl.BlockSpec((_ROWS, k), lambda i: (i, 0)),
            pl.BlockSpec((k, _H), lambda i: (0, 0)),
        ],
        out_specs=[
            pl.BlockSpec((_ROWS, _H), lambda i: (i, 0)),
            pl.BlockSpec((_ROWS, _DVW), lambda i: (i, 0)),
        ],
        out_shape=[
            jax.ShapeDtypeStruct((_N, _H), jnp.float32),
            jax.ShapeDtypeStruct((_N, _DVW), jnp.float32),
        ],
    )(deg_parts, x, w)


def _layer2_body(acc_ref, g_ref, dinv_ref, b1_ref, w2_ref, g2_ref):
    dinv = dinv_ref[:, 0:1]
    pre = (acc_ref[0] + acc_ref[1] + g_ref[...]) * dinv + b1_ref[...]
    h1 = jnp.maximum(pre, 0.0)
    g2_ref[...] = jnp.dot(h1, w2_ref[...],
                          preferred_element_type=jnp.float32) * dinv


def _layer2(acc_parts, g1, dinv, b1, w2):
    return pl.pallas_call(
        _layer2_body,
        grid=(_NBLK,),
        in_specs=[
            pl.BlockSpec((_NC, _ROWS, _H), lambda i: (0, i, 0)),
            pl.BlockSpec((_ROWS, _H), lambda i: (i, 0)),
            pl.BlockSpec((_ROWS, _DVW), lambda i: (i, 0)),
            pl.BlockSpec((1, _H), lambda i: (0, 0)),
            pl.BlockSpec((_H, _H), lambda i: (0, 0)),
        ],
        out_specs=pl.BlockSpec((_ROWS, _H), lambda i: (i, 0)),
        out_shape=jax.ShapeDtypeStruct((_N, _H), jnp.float32),
    )(acc_parts, g1, dinv, b1, w2)


def _final_body(acc_ref, g_ref, dinv_ref, b2_ref, batch_ref, wc_ref, bc_ref,
                out_ref, pool_ref):
    i = pl.program_id(0)

    @pl.when(i == 0)
    def _():
        pool_ref[...] = jnp.zeros_like(pool_ref)

    pre = (acc_ref[0] + acc_ref[1] + g_ref[...]) * dinv_ref[:, 0:1] + b2_ref[...]
    h2 = jnp.maximum(pre, 0.0)
    hcat = jnp.concatenate(
        [h2, jnp.ones((_ROWS, 1), jnp.float32),
         jnp.zeros((_ROWS, _G - _H - 1), jnp.float32)], axis=1)
    onehot = (batch_ref[...] ==
              lax.broadcasted_iota(jnp.int32, (1, _G), 1)).astype(jnp.float32)
    pool_ref[...] += lax.dot_general(onehot, hcat, (((0,), (0,)), ((), ())),
                                     preferred_element_type=jnp.float32)

    @pl.when(i == _NBLK - 1)
    def _():
        sums = pool_ref[:, :_H]
        counts = pool_ref[:, _H:_H + 1]
        pooled = sums / jnp.maximum(counts, 1.0)
        out_ref[...] = jnp.dot(pooled, wc_ref[...],
                               preferred_element_type=jnp.float32) + bc_ref[...]


def _final(acc_parts, g2, dinv, b2, batch2d, wc_pad, bc_pad):
    return pl.pallas_call(
        _final_body,
        grid=(_NBLK,),
        in_specs=[
            pl.BlockSpec((_NC, _ROWS, _H), lambda i: (0, i, 0)),
            pl.BlockSpec((_ROWS, _H), lambda i: (i, 0)),
            pl.BlockSpec((_ROWS, _DVW), lambda i: (i, 0)),
            pl.BlockSpec((1, _H), lambda i: (0, 0)),
            pl.BlockSpec((_ROWS, 1), lambda i: (i, 0)),
            pl.BlockSpec((_H, _G), lambda i: (0, 0)),
            pl.BlockSpec((1, _G), lambda i: (0, 0)),
        ],
        out_specs=pl.BlockSpec((_G, _G), lambda i: (0, 0)),
        out_shape=jax.ShapeDtypeStruct((_G, _G), jnp.float32),
        scratch_shapes=[pltpu.VMEM((_G, _G), jnp.float32)],
    )(acc_parts, g2, dinv, b2, batch2d, wc_pad, bc_pad)


def kernel(x, edge_index, batch, W1, b1, W2, b2, Wc, bc):
    ones_c = jnp.ones((_C, _DW), jnp.float32)
    zeros_d = jnp.zeros((_N, _DW), jnp.float32)
    batch2d = batch.reshape(_N, 1)
    b1r = b1.reshape(1, _H)
    b2r = b2.reshape(1, _H)
    wc_pad = jnp.zeros((_H, _G), jnp.float32).at[:, :_OUT].set(Wc)
    bc_pad = jnp.zeros((1, _G), jnp.float32).at[0, :_OUT].set(bc)

    src3 = edge_index[0].reshape(_NW, _NCHUNK, _C)
    dst3 = edge_index[1].reshape(_NW, _NCHUNK, _C)
    deg_parts = _sc_degree(dst3, ones_c, zeros_d)
    g1, dinv = _mm_scale(deg_parts, x, W1)
    acc1 = _sc_aggregate(g1, src3, dst3)
    g2 = _layer2(acc1, g1, dinv, b1r, W2)
    acc2 = _sc_aggregate(g2, src3, dst3)
    outp = _final(acc2, g2, dinv, b2r, batch2d, wc_pad, bc_pad)
    return outp[:, :_OUT]


# speedup vs baseline: 1.2306x; 1.2306x over previous
import functools

import jax
import jax.numpy as jnp
from jax import lax
from jax.experimental import pallas as pl
from jax.experimental.pallas import tpu as pltpu
from jax.experimental.pallas import tpu_sc as plsc

_N = 10000
_E = 320000
_DIN = 128
_H = 64
_G = 128
_OUT = 2

_NC = 2
_NS = 16
_NW = _NC * _NS
_EPW = _E // _NW
_C = 100
_NCHUNK = _EPW // _C
_NB = 4
_RPT = 632
_RLAST = _N - (_NS - 1) * _RPT
_DW = 8
_DVW = 8

_ROWS = 1000
_NBLK = _N // _ROWS

_sc_mesh = plsc.VectorSubcoreMesh(core_axis_name="c", subcore_axis_name="s",
                                  num_cores=_NC, num_subcores=_NS)
_sc_params = pltpu.CompilerParams(use_tc_tiling_on_sc=False)


_DEG_WIN = 8


@functools.partial(
    pl.kernel,
    out_type=jax.ShapeDtypeStruct((_NC, _N, _DW), jnp.float32),
    mesh=_sc_mesh,
    scratch_types=[
        pltpu.VMEM_SHARED((_N, _DW), jnp.float32),
        pltpu.VMEM((_C, _DW), jnp.float32),
        pltpu.VMEM((_NCHUNK, _C), jnp.int32),
        pltpu.SemaphoreType.DMA,
    ],
    compiler_params=_sc_params,
)
def _sc_degree(dst3_hbm, ones_hbm, zeros_hbm, out_hbm, dacc, ones_v, dst_v,
               ssem):
    c = lax.axis_index("c")
    s = lax.axis_index("s")
    wid = s * _NC + c

    idx_cp = pltpu.async_copy(dst3_hbm.at[wid], dst_v, ssem)
    pltpu.sync_copy(ones_hbm, ones_v)

    @pl.when(s < _NS - 1)
    def _():
        pltpu.sync_copy(zeros_hbm.at[pl.ds(s * _RPT, _RPT)],
                        dacc.at[pl.ds(s * _RPT, _RPT)])

    @pl.when(s == _NS - 1)
    def _():
        pltpu.sync_copy(zeros_hbm.at[pl.ds(s * _RPT, _RLAST)],
                        dacc.at[pl.ds(s * _RPT, _RLAST)])

    idx_cp.wait()
    plsc.subcore_barrier()

    def body(j, carry):
        pltpu.async_copy(ones_v, dacc.at[dst_v.at[j]], ssem, add=True)

        @pl.when(j >= _DEG_WIN)
        def _():
            pltpu.make_async_copy(ones_v, dacc.at[dst_v.at[j - _DEG_WIN]],
                                  ssem).wait()

        return carry

    lax.fori_loop(0, _NCHUNK, body, 0)
    for k in range(_DEG_WIN):
        pltpu.make_async_copy(ones_v, dacc.at[dst_v.at[_NCHUNK - _DEG_WIN + k]],
                              ssem).wait()
    plsc.subcore_barrier()

    @pl.when(s < _NS - 1)
    def _():
        pltpu.sync_copy(dacc.at[pl.ds(s * _RPT, _RPT)],
                        out_hbm.at[c, pl.ds(s * _RPT, _RPT)])

    @pl.when(s == _NS - 1)
    def _():
        pltpu.sync_copy(dacc.at[pl.ds(s * _RPT, _RLAST)],
                        out_hbm.at[c, pl.ds(s * _RPT, _RLAST)])


@functools.partial(
    pl.kernel,
    out_type=jax.ShapeDtypeStruct((_NC, _N, _H), jnp.float32),
    mesh=_sc_mesh,
    scratch_types=[
        pltpu.VMEM_SHARED((_N, _H), jnp.float32),
        pltpu.VMEM((_NCHUNK, _C), jnp.int32),
        pltpu.VMEM((_NCHUNK, _C), jnp.int32),
        pltpu.VMEM((_NB, _C, _H), jnp.float32),
        pltpu.VMEM((_RPT, _H), jnp.float32),
    ] + [pltpu.SemaphoreType.DMA] * (2 * _NB),
    compiler_params=_sc_params,
)
def _sc_aggregate(g_hbm, src3_hbm, dst3_hbm, out_hbm, acc, src_v,
                  dst_v, rows_v, zbuf, *sems):
    gsem = sems[:_NB]
    ssem = sems[_NB:]
    c = lax.axis_index("c")
    s = lax.axis_index("s")
    wid = s * _NC + c

    src_cp = pltpu.async_copy(src3_hbm.at[wid], src_v, sems[0])
    dst_cp = pltpu.async_copy(dst3_hbm.at[wid], dst_v, sems[1])

    def zfill(i, carry):
        for k in range(_H // 16):
            zbuf[i, pl.ds(16 * k, 16)] = jnp.zeros((16,), jnp.float32)
        return carry

    lax.fori_loop(0, _RPT, zfill, 0)

    @pl.when(s < _NS - 1)
    def _():
        pltpu.sync_copy(zbuf, acc.at[pl.ds(s * _RPT, _RPT)])

    @pl.when(s == _NS - 1)
    def _():
        pltpu.sync_copy(zbuf.at[pl.ds(0, _RLAST)],
                        acc.at[pl.ds(s * _RPT, _RLAST)])

    src_cp.wait()
    dst_cp.wait()
    plsc.subcore_barrier()

    def g_issue(b, j):
        pltpu.async_copy(g_hbm.at[src_v.at[j]], rows_v.at[b], gsem[b])

    def g_wait(b, j):
        pltpu.make_async_copy(g_hbm.at[src_v.at[j]], rows_v.at[b],
                              gsem[b]).wait()

    def s_issue(b, j):
        pltpu.async_copy(rows_v.at[b], acc.at[dst_v.at[j]], ssem[b], add=True)

    def s_wait(b, j):
        pltpu.make_async_copy(rows_v.at[b], acc.at[dst_v.at[j]],
                              ssem[b]).wait()

    for b in range(_NB - 1):
        g_issue(b, b)
    g_wait(0, 0)
    s_issue(0, 0)
    g_issue(_NB - 1, _NB - 1)
    for b in range(1, _NB):
        g_wait(b, b)
        s_issue(b, b)
        b3 = (b + _NB - 1) % _NB
        s_wait(b3, b - 1)
        g_issue(b3, b + _NB - 1)

    def body(j0, carry):
        base = j0 * _NB
        for b in range(_NB):
            j = base + b
            g_wait(b, j)
            s_issue(b, j)
            b3 = (b + _NB - 1) % _NB
            s_wait(b3, j - 1)
            g_issue(b3, j + _NB - 1)
        return carry

    lax.fori_loop(1, _NCHUNK // _NB - 1, body, 0)

    base = _NCHUNK - _NB
    g_wait(0, base)
    s_issue(0, base)
    s_wait(_NB - 1, base - 1)
    g_issue(_NB - 1, _NCHUNK - 1)
    for b in range(1, _NB):
        g_wait(b, base + b)
        s_issue(b, base + b)
    for b in range(_NB):
        s_wait(b, base + b)

    plsc.subcore_barrier()

    @pl.when(s < _NS - 1)
    def _():
        pltpu.sync_copy(acc.at[pl.ds(s * _RPT, _RPT)],
                        out_hbm.at[c, pl.ds(s * _RPT, _RPT)])

    @pl.when(s == _NS - 1)
    def _():
        pltpu.sync_copy(acc.at[pl.ds(s * _RPT, _RLAST)],
                        out_hbm.at[c, pl.ds(s * _RPT, _RLAST)])


def _mm_scale_body(deg_ref, x_ref, w_ref, g_ref, dinv_ref):
    hw = jnp.dot(x_ref[...], w_ref[...], preferred_element_type=jnp.float32)
    deg = deg_ref[0, :, 0:1] + deg_ref[1, :, 0:1] + 1.0
    dinv = lax.rsqrt(jnp.maximum(deg, 1.0))
    dinv_ref[...] = jnp.broadcast_to(dinv, (_ROWS, _DVW))
    g_ref[...] = hw * dinv


def _mm_scale(deg_parts, x, w):
    k = x.shape[1]
    return pl.pallas_call(
        _mm_scale_body,
        grid=(_NBLK,),
        in_specs=[
            pl.BlockSpec((_NC, _ROWS, _DW), lambda i: (0, i, 0)),
            pl.BlockSpec((_ROWS, k), lambda i: (i, 0)),
            pl.BlockSpec((k, _H), lambda i: (0, 0)),
        ],
        out_specs=[
            pl.BlockSpec((_ROWS, _H), lambda i: (i, 0)),
            pl.BlockSpec((_ROWS, _DVW), lambda i: (i, 0)),
        ],
        out_shape=[
            jax.ShapeDtypeStruct((_N, _H), jnp.float32),
            jax.ShapeDtypeStruct((_N, _DVW), jnp.float32),
        ],
    )(deg_parts, x, w)


def _layer2_body(acc_ref, g_ref, dinv_ref, b1_ref, w2_ref, g2_ref):
    dinv = dinv_ref[:, 0:1]
    pre = (acc_ref[0] + acc_ref[1] + g_ref[...]) * dinv + b1_ref[...]
    h1 = jnp.maximum(pre, 0.0)
    g2_ref[...] = jnp.dot(h1, w2_ref[...],
                          preferred_element_type=jnp.float32) * dinv


def _layer2(acc_parts, g1, dinv, b1, w2):
    return pl.pallas_call(
        _layer2_body,
        grid=(_NBLK,),
        in_specs=[
            pl.BlockSpec((_NC, _ROWS, _H), lambda i: (0, i, 0)),
            pl.BlockSpec((_ROWS, _H), lambda i: (i, 0)),
            pl.BlockSpec((_ROWS, _DVW), lambda i: (i, 0)),
            pl.BlockSpec((1, _H), lambda i: (0, 0)),
            pl.BlockSpec((_H, _H), lambda i: (0, 0)),
        ],
        out_specs=pl.BlockSpec((_ROWS, _H), lambda i: (i, 0)),
        out_shape=jax.ShapeDtypeStruct((_N, _H), jnp.float32),
    )(acc_parts, g1, dinv, b1, w2)


def _final_body(acc_ref, g_ref, dinv_ref, b2_ref, batch_ref, wc_ref, bc_ref,
                out_ref, pool_ref):
    i = pl.program_id(0)

    @pl.when(i == 0)
    def _():
        pool_ref[...] = jnp.zeros_like(pool_ref)

    pre = (acc_ref[0] + acc_ref[1] + g_ref[...]) * dinv_ref[:, 0:1] + b2_ref[...]
    h2 = jnp.maximum(pre, 0.0)
    hcat = jnp.concatenate(
        [h2, jnp.ones((_ROWS, 1), jnp.float32),
         jnp.zeros((_ROWS, _G - _H - 1), jnp.float32)], axis=1)
    onehot = (batch_ref[...] ==
              lax.broadcasted_iota(jnp.int32, (1, _G), 1)).astype(jnp.float32)
    pool_ref[...] += lax.dot_general(onehot, hcat, (((0,), (0,)), ((), ())),
                                     preferred_element_type=jnp.float32)

    @pl.when(i == _NBLK - 1)
    def _():
        sums = pool_ref[:, :_H]
        counts = pool_ref[:, _H:_H + 1]
        pooled = sums / jnp.maximum(counts, 1.0)
        out_ref[...] = jnp.dot(pooled, wc_ref[...],
                               preferred_element_type=jnp.float32) + bc_ref[...]


def _final(acc_parts, g2, dinv, b2, batch2d, wc_pad, bc_pad):
    return pl.pallas_call(
        _final_body,
        grid=(_NBLK,),
        in_specs=[
            pl.BlockSpec((_NC, _ROWS, _H), lambda i: (0, i, 0)),
            pl.BlockSpec((_ROWS, _H), lambda i: (i, 0)),
            pl.BlockSpec((_ROWS, _DVW), lambda i: (i, 0)),
            pl.BlockSpec((1, _H), lambda i: (0, 0)),
            pl.BlockSpec((_ROWS, 1), lambda i: (i, 0)),
            pl.BlockSpec((_H, _G), lambda i: (0, 0)),
            pl.BlockSpec((1, _G), lambda i: (0, 0)),
        ],
        out_specs=pl.BlockSpec((_G, _G), lambda i: (0, 0)),
        out_shape=jax.ShapeDtypeStruct((_G, _G), jnp.float32),
        scratch_shapes=[pltpu.VMEM((_G, _G), jnp.float32)],
    )(acc_parts, g2, dinv, b2, batch2d, wc_pad, bc_pad)


def kernel(x, edge_index, batch, W1, b1, W2, b2, Wc, bc):
    ones_c = jnp.ones((_C, _DW), jnp.float32)
    zeros_d = jnp.zeros((_N, _DW), jnp.float32)
    batch2d = batch.reshape(_N, 1)
    b1r = b1.reshape(1, _H)
    b2r = b2.reshape(1, _H)
    wc_pad = jnp.zeros((_H, _G), jnp.float32).at[:, :_OUT].set(Wc)
    bc_pad = jnp.zeros((1, _G), jnp.float32).at[0, :_OUT].set(bc)

    src3 = edge_index[0].reshape(_NW, _NCHUNK, _C)
    dst3 = edge_index[1].reshape(_NW, _NCHUNK, _C)
    deg_parts = _sc_degree(dst3, ones_c, zeros_d)
    g1, dinv = _mm_scale(deg_parts, x, W1)
    acc1 = _sc_aggregate(g1, src3, dst3)
    g2 = _layer2(acc1, g1, dinv, b1r, W2)
    acc2 = _sc_aggregate(g2, src3, dst3)
    outp = _final(acc2, g2, dinv, b2r, batch2d, wc_pad, bc_pad)
    return outp[:, :_OUT]
